# baseline (speedup 1.0000x reference)
import jax
import jax.numpy as jnp
from jax import lax
from jax.experimental import pallas as pl
from jax.experimental.pallas import tpu as pltpu
from jax.experimental.pallas import tpu_sc as plsc

B = 16384
FACTOR = 32
ENV_NUM = 4
NC = 2
NS = 16
L = 16
NW = NC * NS
CHUNK = B // NW
NBLK = CHUNK // L
GCH = 128
NG = CHUNK // GCH

_LC0 = -0.76336156
_LC1 = 0.9123227
_LC2 = -0.09557938


def _sc_body(users_hbm, items_hbm, envs_hbm,
             u_inv_hbm, i_inv_hbm, u_env_hbm, i_env_hbm,
             env_hbm, w_hbm, b_hbm,
             inv_hbm, envsc_hbm, lsm_hbm,
             uidx_v, iidx_v, envs_v,
             uinv_v, iinv_v, uenv_v, ienv_v,
             envt_v, w_v, b_v,
             inv_v, envsc_v, lsm_v, sem):
    wid = lax.axis_index("s") * NC + lax.axis_index("c")
    base = wid * CHUNK
    row4 = wid * NG

    pltpu.sync_copy(users_hbm.at[pl.ds(row4, NG)], uidx_v)
    pltpu.sync_copy(items_hbm.at[pl.ds(row4, NG)], iidx_v)
    pltpu.sync_copy(envs_hbm.at[pl.ds(base, CHUNK)], envs_v)
    pltpu.sync_copy(env_hbm, envt_v)
    pltpu.sync_copy(w_hbm, w_v)
    pltpu.sync_copy(b_hbm, b_v)

    copies = []
    for j in range(NG):
        dst = pl.ds(j * GCH, GCH)
        copies.append(pltpu.async_copy(u_inv_hbm.at[uidx_v.at[j]], uinv_v.at[dst], sem))
        copies.append(pltpu.async_copy(i_inv_hbm.at[iidx_v.at[j]], iinv_v.at[dst], sem))
        copies.append(pltpu.async_copy(u_env_hbm.at[uidx_v.at[j]], uenv_v.at[dst], sem))
        copies.append(pltpu.async_copy(i_env_hbm.at[iidx_v.at[j]], ienv_v.at[dst], sem))
    for c in copies:
        c.wait()

    wrows = [w_v[pl.ds(16 * r, L)] for r in range(2 * ENV_NUM)]
    bvec = b_v[pl.ds(0, L)]

    def blk_body(blk, carry):
        o = blk * L
        rows = o + lax.iota(jnp.int32, L)
        env16 = envs_v[pl.ds(o, L)]
        zero = jnp.zeros((L,), jnp.float32)
        acc_i = zero
        acc_e = zero
        l0 = zero
        l1 = zero
        l2 = zero
        l3 = zero
        envbase = env16 * FACTOR
        for f in range(FACTOR):
            fs = jnp.full((L,), f, jnp.int32)
            u = plsc.load_gather(uinv_v, [rows, fs])
            it = plsc.load_gather(iinv_v, [rows, fs])
            p = u * it
            acc_i = acc_i + p
            ue = plsc.load_gather(uenv_v, [rows, fs])
            ie = plsc.load_gather(ienv_v, [rows, fs])
            ee = plsc.load_gather(envt_v, [envbase + f])
            acc_e = acc_e + ue * ie * ee
            h, lane = divmod(f, L)
            l0 = l0 + p * wrows[0 + h][lane]
            l1 = l1 + p * wrows[2 + h][lane]
            l2 = l2 + p * wrows[4 + h][lane]
            l3 = l3 + p * wrows[6 + h][lane]
        inv_v[pl.ds(o, L)] = acc_i
        envsc_v[pl.ds(o, L)] = acc_i + acc_e
        l0 = l0 + bvec[0]
        l1 = l1 + bvec[1]
        l2 = l2 + bvec[2]
        l3 = l3 + bvec[3]
        m = jnp.maximum(jnp.maximum(l0, l1), jnp.maximum(l2, l3))
        t0 = jnp.exp(l0 - m)
        t1 = jnp.exp(l1 - m)
        t2 = jnp.exp(l2 - m)
        t3 = jnp.exp(l3 - m)
        s = t0 + t1 + t2 + t3
        y = _LC0 + s * (_LC1 + _LC2 * s)
        y = y + s * jnp.exp(-y) - 1.0
        y = y + s * jnp.exp(-y) - 1.0
        shift = m + y
        lsmbase = rows * ENV_NUM
        plsc.store_scatter(lsm_v, [lsmbase], l0 - shift)
        plsc.store_scatter(lsm_v, [lsmbase + 1], l1 - shift)
        plsc.store_scatter(lsm_v, [lsmbase + 2], l2 - shift)
        plsc.store_scatter(lsm_v, [lsmbase + 3], l3 - shift)
        return carry

    lax.fori_loop(0, NBLK, blk_body, 0)

    pltpu.sync_copy(inv_v, inv_hbm.at[pl.ds(base, CHUNK)])
    pltpu.sync_copy(envsc_v, envsc_hbm.at[pl.ds(base, CHUNK)])
    pltpu.sync_copy(lsm_v, lsm_hbm.at[pl.ds(base * ENV_NUM, CHUNK * ENV_NUM)])


_sc_call = pl.kernel(
    _sc_body,
    out_type=(
        jax.ShapeDtypeStruct((B,), jnp.float32),
        jax.ShapeDtypeStruct((B,), jnp.float32),
        jax.ShapeDtypeStruct((B * ENV_NUM,), jnp.float32),
    ),
    mesh=plsc.VectorSubcoreMesh(core_axis_name="c", subcore_axis_name="s"),
    scratch_types=[
        pltpu.VMEM((NG, GCH), jnp.int32),
        pltpu.VMEM((NG, GCH), jnp.int32),
        pltpu.VMEM((CHUNK,), jnp.int32),
        pltpu.VMEM((CHUNK, FACTOR), jnp.float32),
        pltpu.VMEM((CHUNK, FACTOR), jnp.float32),
        pltpu.VMEM((CHUNK, FACTOR), jnp.float32),
        pltpu.VMEM((CHUNK, FACTOR), jnp.float32),
        pltpu.VMEM((ENV_NUM * FACTOR,), jnp.float32),
        pltpu.VMEM((ENV_NUM * FACTOR,), jnp.float32),
        pltpu.VMEM((L,), jnp.float32),
        pltpu.VMEM((CHUNK,), jnp.float32),
        pltpu.VMEM((CHUNK,), jnp.float32),
        pltpu.VMEM((CHUNK * ENV_NUM,), jnp.float32),
        pltpu.SemaphoreType.DMA,
    ],
    compiler_params=pltpu.CompilerParams(
        use_tc_tiling_on_sc=False, needs_layout_passes=False),
)


def kernel(users_id, items_id, envs_id, alpha, emb_user_inv, emb_item_inv,
           emb_user_env, emb_item_env, emb_env, W, b):
    del alpha
    users2 = users_id.reshape(NW * NG, GCH)
    items2 = items_id.reshape(NW * NG, GCH)
    w_flat = W.reshape(ENV_NUM * FACTOR)
    b_pad = jnp.pad(b, (0, L - ENV_NUM))
    env_flat = emb_env.reshape(ENV_NUM * FACTOR)
    inv_score, env_score, env_outputs = _sc_call(
        users2, items2, envs_id,
        emb_user_inv, emb_item_inv, emb_user_env, emb_item_env,
        env_flat, w_flat, b_pad)
    return inv_score, env_score, env_outputs.reshape(B, ENV_NUM)
